# baseline (speedup 1.0000x reference)
import jax
import jax.numpy as jnp
from jax.experimental import pallas as pl
from jax.experimental.pallas import tpu as pltpu

B_V, L_V, C = 32, 197, 512
L_S = 196
B_T, L_T = 32, 50
HID = 102
K = 47
NUM_KEEP = 118
NEG = -1e30


B_CHUNK = 8


def _prep_kernel(x_ref, cls_ref, cap_ref, ln_g_ref, ln_b_ref,
                 w1_ref, b1_ref, w2_ref, b2_ref, scale_ref,
                 logits_ref, score_ref, g0_ref, capn_ref):
    x = x_ref[:]
    invn = jax.lax.rsqrt(jnp.maximum(jnp.sum(x * x, axis=-1), 1e-24))
    cls = cls_ref[:]
    g0 = cls * jax.lax.rsqrt(
        jnp.maximum(jnp.sum(cls * cls, axis=-1, keepdims=True), 1e-24))
    g0_ref[:] = g0
    sattn = jnp.sum(g0[:, None, :] * x, axis=-1) * invn
    cap = cap_ref[:]
    capn = cap * jax.lax.rsqrt(
        jnp.maximum(jnp.sum(cap * cap, axis=-1, keepdims=True), 1e-24))

    @pl.when(pl.program_id(0) == 0)
    def _():
        capn_ref[:] = capn

    cap0 = capn[:, 0, :]
    for i in range(B_CHUNK):
        ci = jax.lax.dot_general(cap0, x[i], (((1,), (1,)), ((), ())),
                                 precision=jax.lax.Precision.HIGHEST,
                                 preferred_element_type=jnp.float32)
        score_ref[:, i, :] = ci * invn[i:i + 1, :] + sattn[i:i + 1, :]
    m = jnp.mean(x, axis=-1, keepdims=True)
    xc = x - m
    v = jnp.mean(xc * xc, axis=-1, keepdims=True)
    h = xc * jax.lax.rsqrt(v + 1e-5) * ln_g_ref[:] + ln_b_ref[:]
    h2 = h.reshape(B_CHUNK * L_S, C)
    h1 = jax.lax.dot_general(h2, w1_ref[:], (((1,), (0,)), ((), ())),
                             preferred_element_type=jnp.float32) + b1_ref[:]
    h1 = 0.5 * h1 * (1.0 + jax.lax.erf(h1 * 0.7071067811865476))
    lg = jax.lax.dot_general(h1, w2_ref[:], (((1,), (0,)), ((), ())),
                             preferred_element_type=jnp.float32) + b2_ref[:]
    logits_ref[:] = (lg * scale_ref[0, 0]).reshape(B_CHUNK, L_S, K)


def _main_kernel(x_ref, logits_ref, score_ref, g0_ref, capn_ref,
                 wmask_ref, out_ref):
    score = score_ref[0]
    bits = jax.lax.bitcast_convert_type(score, jnp.int32)
    key = jnp.where(bits < 0, bits ^ jnp.int32(0x7FFFFFFF), bits)
    keepf = float(NUM_KEEP)
    cnt0 = jnp.sum((key >= 0).astype(jnp.float32), axis=1, keepdims=True)
    thr = jnp.where(cnt0 >= keepf, jnp.int32(0), jnp.int32(-2147483648))
    for i in range(30, -1, -1):
        cand = thr | jnp.int32(1 << i)
        c = jnp.sum((key >= cand).astype(jnp.float32), axis=1, keepdims=True)
        thr = jnp.where(c >= keepf, cand, thr)
    gt = key > thr
    eq = key == thr
    eqf = eq.astype(jnp.float32)
    ng = jnp.sum(gt.astype(jnp.float32), axis=1, keepdims=True)
    im = jax.lax.broadcasted_iota(jnp.int32, (L_S, L_S), 0)
    il = jax.lax.broadcasted_iota(jnp.int32, (L_S, L_S), 1)
    tri = (im < il).astype(jnp.float32)
    cum = jax.lax.dot_general(eqf, tri, (((1,), (0,)), ((), ())),
                              preferred_element_type=jnp.float32)
    keep = gt | (eq & (cum < (keepf - ng)))
    keep3 = keep.astype(jnp.float32)[:, :, None] > 0.5
    wl = jnp.where(keep3, logits_ref[:], NEG)
    mx = jnp.max(wl, axis=1, keepdims=True)
    e = jnp.exp(wl - mx)
    s2 = jnp.where(keep, NEG, score)
    mx2 = jnp.max(s2, axis=1, keepdims=True)
    e2 = jnp.exp(s2 - mx2)
    ef = jnp.concatenate([e, e2[:, :, None]], axis=2)
    num = jax.lax.dot_general(ef, x_ref[:], (((1,), (1,)), ((0,), (0,))),
                              preferred_element_type=jnp.float32)
    aggn = num * jax.lax.rsqrt(
        jnp.maximum(jnp.sum(num * num, axis=-1, keepdims=True), 1e-30))
    capn = capn_ref[0]
    sim = jax.lax.dot_general(aggn.reshape(B_V * (K + 1), C), capn,
                              (((1,), (1,)), ((), ())),
                              preferred_element_type=jnp.float32)
    best = jnp.max(sim.reshape(B_V, K + 1, L_T), axis=1)
    sim_g = jax.lax.dot_general(g0_ref[:], capn, (((1,), (1,)), ((), ())),
                                preferred_element_type=jnp.float32)
    best = jnp.maximum(best, sim_g)
    out_ref[0, 0, :] = jnp.sum(best * wmask_ref[0], axis=1)


def kernel(img_embs, cap_embs, cap_lens, ln_g, ln_b, w1, b1, w2, b2, scale):
    spatial = img_embs[:, 1:, :]
    cls = img_embs[:, 0, :]
    nw = cap_lens.astype(jnp.float32)
    wmask = jnp.where(jnp.arange(L_T)[None, :] < cap_lens[:, None],
                      1.0 / nw[:, None], 0.0).reshape(B_T, 1, L_T)

    logits, score, g0, capn = pl.pallas_call(
        _prep_kernel,
        grid=(B_V // B_CHUNK,),
        in_specs=[
            pl.BlockSpec((B_CHUNK, L_S, C), lambda b: (b, 0, 0)),
            pl.BlockSpec((B_CHUNK, C), lambda b: (b, 0)),
            pl.BlockSpec((B_T, L_T, C), lambda b: (0, 0, 0)),
            pl.BlockSpec((1, C), lambda b: (0, 0)),
            pl.BlockSpec((1, C), lambda b: (0, 0)),
            pl.BlockSpec((C, HID), lambda b: (0, 0)),
            pl.BlockSpec((1, HID), lambda b: (0, 0)),
            pl.BlockSpec((HID, K), lambda b: (0, 0)),
            pl.BlockSpec((1, K), lambda b: (0, 0)),
            pl.BlockSpec((1, 1), lambda b: (0, 0)),
        ],
        out_specs=(
            pl.BlockSpec((B_CHUNK, L_S, K), lambda b: (b, 0, 0)),
            pl.BlockSpec((B_T, B_CHUNK, L_S), lambda b: (0, b, 0)),
            pl.BlockSpec((B_CHUNK, C), lambda b: (b, 0)),
            pl.BlockSpec((B_T, L_T, C), lambda b: (0, 0, 0)),
        ),
        out_shape=(
            jax.ShapeDtypeStruct((B_V, L_S, K), jnp.float32),
            jax.ShapeDtypeStruct((B_T, B_V, L_S), jnp.float32),
            jax.ShapeDtypeStruct((B_V, C), jnp.float32),
            jax.ShapeDtypeStruct((B_T, L_T, C), jnp.float32),
        ),
    )(spatial, cls, cap_embs, ln_g.reshape(1, C), ln_b.reshape(1, C),
      w1, b1.reshape(1, HID), w2, b2.reshape(1, K), scale.reshape(1, 1))

    out = pl.pallas_call(
        _main_kernel,
        grid=(B_T,),
        in_specs=[
            pl.BlockSpec((B_V, L_S, C), lambda t: (0, 0, 0)),
            pl.BlockSpec((B_V, L_S, K), lambda t: (0, 0, 0)),
            pl.BlockSpec((1, B_V, L_S), lambda t: (t, 0, 0)),
            pl.BlockSpec((B_V, C), lambda t: (0, 0)),
            pl.BlockSpec((1, L_T, C), lambda t: (t, 0, 0)),
            pl.BlockSpec((1, 1, L_T), lambda t: (t, 0, 0)),
        ],
        out_specs=pl.BlockSpec((1, 1, B_V), lambda t: (t, 0, 0)),
        out_shape=jax.ShapeDtypeStruct((B_T, 1, B_V), jnp.float32),
        compiler_params=pltpu.CompilerParams(
            dimension_semantics=("parallel",)),
    )(spatial, logits, score, g0, capn, wmask)

    return out.reshape(B_T, B_V).T


# speedup vs baseline: 1.8854x; 1.8854x over previous
import jax
import jax.numpy as jnp
from jax.experimental import pallas as pl
from jax.experimental.pallas import tpu as pltpu

B_V, L_V, C = 32, 197, 512
L_S = 196
B_T, L_T = 32, 50
HID = 102
K = 47
NUM_KEEP = 118
NEG = -1e30


B_CHUNK = 8


def _prep_kernel(x_ref, cls_ref, cap_ref, ln_g_ref, ln_b_ref,
                 w1_ref, b1_ref, w2_ref, b2_ref, scale_ref,
                 logits_ref, score_ref, keep_ref, g0_ref, capn_ref):
    x = x_ref[:]
    invn = jax.lax.rsqrt(jnp.maximum(jnp.sum(x * x, axis=-1), 1e-24))
    cls = cls_ref[:]
    g0 = cls * jax.lax.rsqrt(
        jnp.maximum(jnp.sum(cls * cls, axis=-1, keepdims=True), 1e-24))
    g0_ref[:] = g0
    sattn = jnp.sum(g0[:, None, :] * x, axis=-1) * invn
    @pl.when(pl.program_id(0) == 0)
    def _():
        cap = cap_ref[:]
        capn_ref[:] = cap * jax.lax.rsqrt(
            jnp.maximum(jnp.sum(cap * cap, axis=-1, keepdims=True), 1e-24))

    c0 = cap_ref[:, 0, :]
    cap0 = c0 * jax.lax.rsqrt(
        jnp.maximum(jnp.sum(c0 * c0, axis=-1, keepdims=True), 1e-24))
    for i in range(B_CHUNK):
        ci = jax.lax.dot_general(cap0, x[i], (((1,), (1,)), ((), ())),
                                 precision=jax.lax.Precision.HIGHEST,
                                 preferred_element_type=jnp.float32)
        score_ref[:, i, :] = ci * invn[i:i + 1, :] + sattn[i:i + 1, :]
    scores = score_ref[:]
    sbits = jax.lax.bitcast_convert_type(scores, jnp.int32)
    skey = jnp.where(sbits < 0, sbits ^ jnp.int32(0x7FFFFFFF), sbits)
    keepf = float(NUM_KEEP)
    cnt0 = jnp.sum((skey >= 0).astype(jnp.float32), axis=2, keepdims=True)
    thr = jnp.where(cnt0 >= keepf, jnp.int32(0), jnp.int32(-2147483648))
    for i in range(30, -1, -1):
        cand = thr | jnp.int32(1 << i)
        c = jnp.sum((skey >= cand).astype(jnp.float32), axis=2, keepdims=True)
        thr = jnp.where(c >= keepf, cand, thr)
    gt = skey > thr
    eq = skey == thr
    eqf = eq.astype(jnp.float32)
    ng = jnp.sum(gt.astype(jnp.float32), axis=2, keepdims=True)
    im = jax.lax.broadcasted_iota(jnp.int32, (L_S, L_S), 0)
    il = jax.lax.broadcasted_iota(jnp.int32, (L_S, L_S), 1)
    tri = (im < il).astype(jnp.float32)
    cum = jax.lax.dot_general(eqf, tri, (((2,), (0,)), ((), ())),
                              preferred_element_type=jnp.float32)
    keep_ref[:] = (gt | (eq & (cum < (keepf - ng)))).astype(jnp.float32)
    m = jnp.mean(x, axis=-1, keepdims=True)
    xc = x - m
    v = jnp.mean(xc * xc, axis=-1, keepdims=True)
    h = xc * jax.lax.rsqrt(v + 1e-5) * ln_g_ref[:] + ln_b_ref[:]
    sc = scale_ref[0, 0]
    for i in range(B_CHUNK):
        h1 = jax.lax.dot_general(h[i], w1_ref[:], (((1,), (0,)), ((), ())),
                                 preferred_element_type=jnp.float32) + b1_ref[:]
        h1 = 0.5 * h1 * (1.0 + jax.lax.erf(h1 * 0.7071067811865476))
        lgt = jax.lax.dot_general(w2_ref[:], h1, (((0,), (1,)), ((), ())),
                                  preferred_element_type=jnp.float32)
        logits_ref[i, :K, :] = (lgt + b2_ref[:]) * sc
        logits_ref[i, K:K + 1, :] = jnp.zeros((1, L_S), jnp.float32)


T_CHUNK = 2


def _main_kernel(x_ref, logits_ref, score_ref, keep_ref, g0_ref, capn_ref,
                 wmask_ref, out_ref):
    x = x_ref[:]
    lg = logits_ref[:]
    is47 = (jax.lax.broadcasted_iota(jnp.int32, (1, K + 1, 1), 1)
            == K).astype(jnp.float32)
    for j in range(T_CHUNK):
        score = score_ref[j]
        m3 = keep_ref[j][:, None, :]
        pen = ((m3 - 1.0) - is47 * (2.0 * m3 - 1.0)) * 1e30
        wl = lg + is47 * score[:, None, :] + pen
        mx = jnp.max(wl, axis=2, keepdims=True)
        ef = jnp.exp(wl - mx)
        num = jax.lax.dot_general(ef, x, (((2,), (1,)), ((0,), (0,))),
                                  preferred_element_type=jnp.float32)
        aggn = num * jax.lax.rsqrt(
            jnp.maximum(jnp.sum(num * num, axis=-1, keepdims=True), 1e-30))
        capn = capn_ref[j]
        sim = jax.lax.dot_general(aggn.reshape(B_V * (K + 1), C), capn,
                                  (((1,), (1,)), ((), ())),
                                  preferred_element_type=jnp.float32)
        best = jnp.max(sim.reshape(B_V, K + 1, L_T), axis=1)
        sim_g = jax.lax.dot_general(g0_ref[:], capn, (((1,), (1,)), ((), ())),
                                    preferred_element_type=jnp.float32)
        best = jnp.maximum(best, sim_g)
        out_ref[j, 0, :] = jnp.sum(best * wmask_ref[j], axis=1)


def kernel(img_embs, cap_embs, cap_lens, ln_g, ln_b, w1, b1, w2, b2, scale):
    spatial = img_embs[:, 1:, :]
    cls = img_embs[:, 0, :]
    nw = cap_lens.astype(jnp.float32)
    wmask = jnp.where(jnp.arange(L_T)[None, :] < cap_lens[:, None],
                      1.0 / nw[:, None], 0.0).reshape(B_T, 1, L_T)

    logits, score, keepm, g0, capn = pl.pallas_call(
        _prep_kernel,
        grid=(B_V // B_CHUNK,),
        in_specs=[
            pl.BlockSpec((B_CHUNK, L_S, C), lambda b: (b, 0, 0)),
            pl.BlockSpec((B_CHUNK, C), lambda b: (b, 0)),
            pl.BlockSpec((B_T, L_T, C), lambda b: (0, 0, 0)),
            pl.BlockSpec((1, C), lambda b: (0, 0)),
            pl.BlockSpec((1, C), lambda b: (0, 0)),
            pl.BlockSpec((C, HID), lambda b: (0, 0)),
            pl.BlockSpec((1, HID), lambda b: (0, 0)),
            pl.BlockSpec((HID, K), lambda b: (0, 0)),
            pl.BlockSpec((K, 1), lambda b: (0, 0)),
            pl.BlockSpec((1, 1), lambda b: (0, 0)),
        ],
        out_specs=(
            pl.BlockSpec((B_CHUNK, K + 1, L_S), lambda b: (b, 0, 0)),
            pl.BlockSpec((B_T, B_CHUNK, L_S), lambda b: (0, b, 0)),
            pl.BlockSpec((B_T, B_CHUNK, L_S), lambda b: (0, b, 0)),
            pl.BlockSpec((B_CHUNK, C), lambda b: (b, 0)),
            pl.BlockSpec((B_T, L_T, C), lambda b: (0, 0, 0)),
        ),
        out_shape=(
            jax.ShapeDtypeStruct((B_V, K + 1, L_S), jnp.float32),
            jax.ShapeDtypeStruct((B_T, B_V, L_S), jnp.float32),
            jax.ShapeDtypeStruct((B_T, B_V, L_S), jnp.float32),
            jax.ShapeDtypeStruct((B_V, C), jnp.float32),
            jax.ShapeDtypeStruct((B_T, L_T, C), jnp.float32),
        ),
    )(spatial, cls, cap_embs, ln_g.reshape(1, C), ln_b.reshape(1, C),
      w1, b1.reshape(1, HID), w2, b2.reshape(K, 1), scale.reshape(1, 1))

    out = pl.pallas_call(
        _main_kernel,
        grid=(B_T // T_CHUNK,),
        in_specs=[
            pl.BlockSpec((B_V, L_S, C), lambda t: (0, 0, 0)),
            pl.BlockSpec((B_V, K + 1, L_S), lambda t: (0, 0, 0)),
            pl.BlockSpec((T_CHUNK, B_V, L_S), lambda t: (t, 0, 0)),
            pl.BlockSpec((T_CHUNK, B_V, L_S), lambda t: (t, 0, 0)),
            pl.BlockSpec((B_V, C), lambda t: (0, 0)),
            pl.BlockSpec((T_CHUNK, L_T, C), lambda t: (t, 0, 0)),
            pl.BlockSpec((T_CHUNK, 1, L_T), lambda t: (t, 0, 0)),
        ],
        out_specs=pl.BlockSpec((T_CHUNK, 1, B_V), lambda t: (t, 0, 0)),
        out_shape=jax.ShapeDtypeStruct((B_T, 1, B_V), jnp.float32),
        compiler_params=pltpu.CompilerParams(
            dimension_semantics=("parallel",)),
    )(spatial, logits, score, keepm, g0, capn, wmask)

    return out.reshape(B_T, B_V).T
